# Initial kernel scaffold; baseline (speedup 1.0000x reference)
import jax
import jax.numpy as jnp
from jax.experimental import pallas as pl


def kernel(x, emb, W1, b1, W2, b2, W3, b3):
    raise NotImplementedError("write your pallas kernel here")


# speedup vs baseline: 4.7440x; 4.7440x over previous
import functools

import jax
import jax.numpy as jnp
from jax import lax
from jax.experimental import pallas as pl
from jax.experimental.pallas import tpu as pltpu
from jax.experimental.pallas import tpu_sc as plsc

VOCAB = 1000000
SEQ = 200
BATCH = 16384
EMB = 3
FEAT = SEQ * EMB

NC = 2
NS = 16
NW = NC * NS
SAMPW = BATCH // NW
SAMP_CHUNK = 32
NCHUNK = SAMPW // SAMP_CHUNK


@functools.cache
def _make_gather():
  mesh = plsc.VectorSubcoreMesh(
      core_axis_name="c", subcore_axis_name="s", num_cores=NC, num_subcores=NS
  )
  n_idx = SAMP_CHUNK * SEQ

  @functools.partial(
      pl.kernel,
      mesh=mesh,
      out_type=jax.ShapeDtypeStruct((BATCH * SEQ, EMB), jnp.float32),
      scratch_types=[
          pltpu.VMEM((n_idx,), jnp.int32),
          pltpu.VMEM((n_idx, 8), jnp.float32),
          pltpu.SemaphoreType.DMA,
      ],
      compiler_params=pltpu.CompilerParams(use_tc_tiling_on_sc=False),
  )
  def gather_kernel(x_hbm, emb_hbm, out_hbm, idx_v, rows_v, sem):
    wid = lax.axis_index("s") * NC + lax.axis_index("c")
    base = wid * SAMPW

    def body(j, _):
      b0 = base + j * SAMP_CHUNK
      pltpu.sync_copy(x_hbm.at[pl.ds(b0 * SEQ, n_idx)], idx_v)
      pltpu.async_copy(emb_hbm.at[idx_v], rows_v, sem).wait()
      pltpu.sync_copy(rows_v.at[:, 0:EMB], out_hbm.at[pl.ds(b0 * SEQ, n_idx), :])
      return 0

    lax.fori_loop(0, NCHUNK, body, 0)

  return gather_kernel


BB = 1024


def _mlp_body(g_ref, w1_ref, b1_ref, w2_ref, b2_ref, w3_ref, b3_ref, o_ref):
  h = g_ref[...]
  h = jnp.dot(h, w1_ref[...], preferred_element_type=jnp.float32) + b1_ref[...]
  h = jnp.maximum(h, 0.0)
  h = jnp.dot(h, w2_ref[...], preferred_element_type=jnp.float32) + b2_ref[...]
  h = jnp.maximum(h, 0.0)
  z = jnp.dot(h, w3_ref[...], preferred_element_type=jnp.float32) + b3_ref[...]
  o_ref[...] = 1.0 / (1.0 + jnp.exp(-z))


def _mlp(g, w1t, b1, w2t, b2, w3t, b3):
  grid = BATCH // BB
  full = lambda shape: pl.BlockSpec(shape, lambda i: (0, 0))
  return pl.pallas_call(
      _mlp_body,
      grid=(grid,),
      in_specs=[
          pl.BlockSpec((BB, FEAT), lambda i: (i, 0)),
          full((FEAT, 10)),
          full((1, 10)),
          full((10, 5)),
          full((1, 5)),
          full((5, 3)),
          full((1, 3)),
      ],
      out_specs=pl.BlockSpec((BB, 3), lambda i: (i, 0)),
      out_shape=jax.ShapeDtypeStruct((BATCH, 3), jnp.float32),
  )(g, w1t, b1, w2t, b2, w3t, b3)


@jax.jit
def kernel(x, emb, W1, b1, W2, b2, W3, b3):
  x_flat = x.astype(jnp.int32).reshape(BATCH * SEQ)
  emb8 = jnp.pad(emb, ((0, 0), (0, 8 - EMB)))
  g = _make_gather()(x_flat, emb8).reshape(BATCH, FEAT)
  return _mlp(
      g,
      W1.T,
      b1.reshape(1, 10),
      W2.T,
      b2.reshape(1, 5),
      W3.T,
      b3.reshape(1, 3),
  )


# speedup vs baseline: 91.9430x; 19.3808x over previous
import functools

import jax
import jax.numpy as jnp
from jax import lax
from jax.experimental import pallas as pl
from jax.experimental.pallas import tpu as pltpu
from jax.experimental.pallas import tpu_sc as plsc

VOCAB = 1000000
SEQ = 200
BATCH = 16384
EMB = 3
TOTAL = BATCH * SEQ

NC = 2
NS = 16
NW = NC * NS
PER_W = TOTAL // NW
CHUNK = 6400
NCHUNK = PER_W // CHUNK
KSUB = 8
SUB = CHUNK // KSUB


@functools.cache
def _make_gather():
  mesh = plsc.VectorSubcoreMesh(
      core_axis_name="c", subcore_axis_name="s", num_cores=NC, num_subcores=NS
  )
  out1d = jax.ShapeDtypeStruct((TOTAL,), jnp.float32)

  @functools.partial(
      pl.kernel,
      mesh=mesh,
      out_type=(out1d, out1d, out1d),
      scratch_types=[
          pltpu.VMEM((CHUNK,), jnp.int32),
          pltpu.VMEM((CHUNK,), jnp.float32),
          pltpu.VMEM((CHUNK,), jnp.float32),
          pltpu.VMEM((CHUNK,), jnp.float32),
          pltpu.SemaphoreType.DMA,
      ],
      compiler_params=pltpu.CompilerParams(use_tc_tiling_on_sc=False),
  )
  def gather_kernel(x_hbm, t0_hbm, t1_hbm, t2_hbm,
                    g0_hbm, g1_hbm, g2_hbm, idx_v, v0, v1, v2, sem):
    wid = lax.axis_index("s") * NC + lax.axis_index("c")
    base = wid * PER_W
    tabs = (t0_hbm, t1_hbm, t2_hbm)
    vals = (v0, v1, v2)

    def body(j, _):
      o = base + j * CHUNK
      pltpu.sync_copy(x_hbm.at[pl.ds(o, CHUNK)], idx_v)
      cps = []
      for i in range(KSUB):
        sub_idx = idx_v.at[pl.ds(i * SUB, SUB)]
        for d in range(EMB):
          cps.append(
              pltpu.async_copy(
                  tabs[d].at[sub_idx],
                  vals[d].at[pl.ds(i * SUB, SUB)],
                  sem,
              )
          )
      for cp in cps:
        cp.wait()
      for d in range(EMB):
        pltpu.sync_copy(vals[d], (g0_hbm, g1_hbm, g2_hbm)[d].at[pl.ds(o, CHUNK)])
      return 0

    lax.fori_loop(0, NCHUNK, body, 0)

  return gather_kernel


BB = 1024


def _mlp_body(g0_ref, g1_ref, g2_ref, w10_ref, w11_ref, w12_ref, b1_ref,
              w2_ref, b2_ref, w3_ref, b3_ref, o_ref):
  h = jnp.dot(g0_ref[...], w10_ref[...], preferred_element_type=jnp.float32)
  h += jnp.dot(g1_ref[...], w11_ref[...], preferred_element_type=jnp.float32)
  h += jnp.dot(g2_ref[...], w12_ref[...], preferred_element_type=jnp.float32)
  h = jnp.maximum(h + b1_ref[...], 0.0)
  h = jnp.dot(h, w2_ref[...], preferred_element_type=jnp.float32) + b2_ref[...]
  h = jnp.maximum(h, 0.0)
  z = jnp.dot(h, w3_ref[...], preferred_element_type=jnp.float32) + b3_ref[...]
  o_ref[...] = 1.0 / (1.0 + jnp.exp(-z))


def _mlp(g0, g1, g2, w10, w11, w12, b1, w2t, b2, w3t, b3):
  grid = BATCH // BB
  gspec = pl.BlockSpec((BB, SEQ), lambda i: (i, 0))
  full = lambda shape: pl.BlockSpec(shape, lambda i: (0, 0))
  return pl.pallas_call(
      _mlp_body,
      grid=(grid,),
      in_specs=[
          gspec, gspec, gspec,
          full((SEQ, 10)), full((SEQ, 10)), full((SEQ, 10)),
          full((1, 10)),
          full((10, 5)),
          full((1, 5)),
          full((5, 3)),
          full((1, 3)),
      ],
      out_specs=pl.BlockSpec((BB, 3), lambda i: (i, 0)),
      out_shape=jax.ShapeDtypeStruct((BATCH, 3), jnp.float32),
  )(g0, g1, g2, w10, w11, w12, b1, w2t, b2, w3t, b3)


@jax.jit
def kernel(x, emb, W1, b1, W2, b2, W3, b3):
  x_flat = x.astype(jnp.int32).reshape(TOTAL)
  t0 = jnp.asarray(emb[:, 0])
  t1 = jnp.asarray(emb[:, 1])
  t2 = jnp.asarray(emb[:, 2])
  g0, g1, g2 = _make_gather()(x_flat, t0, t1, t2)
  w1r = W1.reshape(10, SEQ, EMB)
  return _mlp(
      g0.reshape(BATCH, SEQ),
      g1.reshape(BATCH, SEQ),
      g2.reshape(BATCH, SEQ),
      w1r[:, :, 0].T,
      w1r[:, :, 1].T,
      w1r[:, :, 2].T,
      b1.reshape(1, 10),
      W2.T,
      b2.reshape(1, 5),
      W3.T,
      b3.reshape(1, 3),
  )
